# baseline (speedup 1.0000x reference)
import functools

import jax
import jax.numpy as jnp
from jax import lax
from jax.experimental import pallas as pl
from jax.experimental.pallas import tpu as pltpu
from jax.experimental.pallas import tpu_sc as plsc

_H = 128
_W = 128
_B = 2
_N = 1024
_P = 32
_QMAX = 30.0

_mesh = plsc.VectorSubcoreMesh(core_axis_name="c", subcore_axis_name="s")


@functools.partial(
    pl.kernel,
    out_type=jax.ShapeDtypeStruct((32, 4 * _P * _P), jnp.float32),
    mesh=_mesh,
    scratch_types=[
        pltpu.VMEM((_N * 16,), jnp.float32),
        pltpu.VMEM((_P * _P,), jnp.float32),
        pltpu.VMEM((4 * _P * _P,), jnp.float32),
    ],
)
def _sc_render(pk_hbm, out_hbm, pk, t, acc):
    wid = lax.axis_index("s") * 2 + lax.axis_index("c")
    b = wid // 16
    pidx = wid % 16
    pr0 = (pidx // 4) * _P
    pc0 = (pidx % 4) * _P
    pr0f = pr0.astype(jnp.float32)
    pc0f = pc0.astype(jnp.float32)
    myb = b.astype(jnp.float32)

    pltpu.sync_copy(pk_hbm, pk)

    ones = jnp.full((16,), 1.0, jnp.float32)
    zeros = jnp.zeros((16,), jnp.float32)

    def init_t(j, c):
        t[pl.ds(j * 16, 16)] = ones
        return c

    lax.fori_loop(0, (_P * _P) // 16, init_t, 0)

    def init_acc(j, c):
        acc[pl.ds(j * 16, 16)] = zeros
        return c

    lax.fori_loop(0, (4 * _P * _P) // 16, init_acc, 0)

    lane = lax.iota(jnp.int32, 16).astype(jnp.float32) + 0.5
    x0v = lane + pc0f
    x1v = x0v + 16.0

    def gauss_body(i, carry):
        v = pk[pl.ds(i * 16, 16)]
        bid = v[0]
        my = v[2]
        ry = v[10]
        ylo_f = jnp.maximum(my - ry - 0.5, pr0f)
        yhi_f = jnp.minimum(my + ry + 0.5, pr0f + (_P - 1.0))

        @pl.when(jnp.logical_and(bid == myb, ylo_f <= yhi_f))
        def _():
            mx = v[1]
            rx = v[11]
            a = v[3]
            bc = v[4]
            d = v[5]
            cr = v[6]
            cg = v[7]
            cb = v[8]
            op = v[9]
            inc0 = jnp.logical_and(mx - rx - 0.5 < pc0f + 16.0,
                                   mx + rx + 0.5 > pc0f)
            inc1 = jnp.logical_and(mx - rx - 0.5 < pc0f + 32.0,
                                   mx + rx + 0.5 > pc0f + 16.0)
            ylo = ylo_f.astype(jnp.int32)
            yhi = yhi_f.astype(jnp.int32) + 1

            def row_body(r, rc):
                dy = r.astype(jnp.float32) + 0.5 - my
                bcdy = bc * dy
                ddy2 = d * dy * dy
                base = (r - pr0) * _P

                def half(bb, xv):
                    dx = xv - mx
                    q = (a * dx + bcdy) * dx + ddy2
                    alpha = op * jnp.exp(q)
                    tt = t[pl.ds(bb, 16)]
                    w = alpha * tt
                    t[pl.ds(bb, 16)] = tt - w
                    o0 = acc[pl.ds(bb, 16)]
                    acc[pl.ds(bb, 16)] = o0 + cr * w
                    o1 = acc[pl.ds(bb + 1024, 16)]
                    acc[pl.ds(bb + 1024, 16)] = o1 + cg * w
                    o2 = acc[pl.ds(bb + 2048, 16)]
                    acc[pl.ds(bb + 2048, 16)] = o2 + cb * w
                    o3 = acc[pl.ds(bb + 3072, 16)]
                    acc[pl.ds(bb + 3072, 16)] = o3 + w

                @pl.when(inc0)
                def _():
                    half(base, x0v)

                @pl.when(inc1)
                def _():
                    half(base + 16, x1v)

                return rc

            lax.fori_loop(ylo, yhi, row_body, 0)

        return carry

    lax.fori_loop(0, _N, gauss_body, 0)

    pltpu.sync_copy(acc, out_hbm.at[wid])


@jax.jit
def kernel(batch_ids, means, sigmas, rhos, colors, opacs):
    c = jnp.cos(rhos[:, 0])
    s = jnp.sin(rhos[:, 0])
    s1 = sigmas[:, 0]
    s2 = sigmas[:, 1]
    i1 = 1.0 / s1
    i2 = 1.0 / s2
    a = -0.5 * (c * c * i1 + s * s * i2)
    d = -0.5 * (s * s * i1 + c * c * i2)
    bc = -0.5 * (2.0 * c * s * (i1 - i2))
    sxx = c * c * s1 + s * s * s2
    syy = s * s * s1 + c * c * s2
    rx = jnp.sqrt(_QMAX * sxx)
    ry = jnp.sqrt(_QMAX * syy)
    z = jnp.zeros_like(a)
    pk = jnp.stack([
        batch_ids.astype(jnp.float32), means[:, 0], means[:, 1],
        a, bc, d, colors[:, 0], colors[:, 1], colors[:, 2],
        opacs[:, 0], ry, rx, z, z, z, z,
    ], axis=1).reshape(-1)
    out = _sc_render(pk)
    out = out.reshape(_B, 4, 4, 4, _P, _P)
    out = out.transpose(0, 3, 1, 4, 2, 5).reshape(_B, 4, _H, _W)
    return out


# speedup vs baseline: 2.4496x; 2.4496x over previous
import functools

import jax
import jax.numpy as jnp
from jax import lax
from jax.experimental import pallas as pl
from jax.experimental.pallas import tpu as pltpu
from jax.experimental.pallas import tpu_sc as plsc

_H = 128
_W = 128
_B = 2
_N = 1024
_P = 32
_QMAX = 30.0

_mesh = plsc.VectorSubcoreMesh(core_axis_name="c", subcore_axis_name="s")


@functools.partial(
    pl.kernel,
    out_type=jax.ShapeDtypeStruct((32, 4 * _P * _P), jnp.float32),
    mesh=_mesh,
    scratch_types=[
        pltpu.VMEM((_N * 16,), jnp.float32),
        pltpu.VMEM((5 * _N,), jnp.float32),
        pltpu.SMEM((_N + 1,), jnp.int32),
        pltpu.VMEM((16,), jnp.int32),
        pltpu.VMEM((_P * _P,), jnp.float32),
        pltpu.VMEM((4 * _P * _P,), jnp.float32),
    ],
)
def _sc_render(pk_hbm, cull_hbm, out_hbm, pk, cull, hits, hvbuf, t, acc):
    wid = lax.axis_index("s") * 2 + lax.axis_index("c")
    b = wid // 16
    pidx = wid % 16
    pr0 = (pidx // 4) * _P
    pc0 = (pidx % 4) * _P
    pr0f = pr0.astype(jnp.float32)
    pc0f = pc0.astype(jnp.float32)
    myb = b.astype(jnp.float32)

    pltpu.sync_copy(pk_hbm, pk)
    pltpu.sync_copy(cull_hbm, cull)

    ones = jnp.full((16,), 1.0, jnp.float32)
    zeros = jnp.zeros((16,), jnp.float32)

    def init_t(j, c):
        t[pl.ds(j * 16, 16)] = ones
        return c

    lax.fori_loop(0, (_P * _P) // 16, init_t, 0)

    def init_acc(j, c):
        acc[pl.ds(j * 16, 16)] = zeros
        return c

    lax.fori_loop(0, (4 * _P * _P) // 16, init_acc, 0)

    lanei = lax.iota(jnp.int32, 16)


    lane = lanei.astype(jnp.float32) + 0.5
    x0v = lane + pc0f
    x1v = x0v + 16.0

    def cull_body(j, cnt):
        j16 = j * 16
        bidv = cull[pl.ds(j16, 16)]
        myv = cull[pl.ds(j16 + _N, 16)]
        ryv = cull[pl.ds(j16 + 2 * _N, 16)]
        mxv = cull[pl.ds(j16 + 3 * _N, 16)]
        rxv = cull[pl.ds(j16 + 4 * _N, 16)]
        hit = bidv == myb
        hit = jnp.logical_and(hit, myv - ryv - 0.5 <= pr0f + (_P - 1.0))
        hit = jnp.logical_and(hit, myv + ryv + 0.5 >= pr0f)
        hit = jnp.logical_and(hit, mxv - rxv - 0.5 < pc0f + float(_P))
        hit = jnp.logical_and(hit, mxv + rxv + 0.5 > pc0f)
        one16 = jnp.full((16,), 1, jnp.int32)
        zero16 = jnp.full((16,), 0, jnp.int32)
        hvbuf[pl.ds(0, 16)] = jnp.where(hit, one16, zero16)
        hb = hvbuf[pl.ds(0, 16)]
        base = j * 16
        for l in range(16):
            hits[cnt] = base + l
            cnt = cnt + hb[l]
        return cnt

    cnt = lax.fori_loop(0, _N // 16, cull_body, 0)

    def hit_body(k, c):
        i = hits[k]
        v = pk[pl.ds(i * 16, 16)]
        mx = v[1]
        my = v[2]
        a = v[3]
        bc = v[4]
        d = v[5]
        cr = v[6]
        cg = v[7]
        cb = v[8]
        lnop = v[9]
        ry = v[10]
        rx = v[11]
        ylo = jnp.maximum(my - ry - 0.5, pr0f).astype(jnp.int32)
        yhi = jnp.minimum(my + ry + 0.5, pr0f + (_P - 1.0)).astype(jnp.int32) + 1
        inc0 = mx - rx - 0.5 < pc0f + 16.0
        inc1 = mx + rx + 0.5 > pc0f + 16.0

        def make_row(xv, off):
            dxh = xv - mx
            u = a * dxh * dxh
            vv = bc * dxh

            def row(r, rc):
                dy = r.astype(jnp.float32) + 0.5 - my
                ddy2 = d * dy * dy + lnop
                q = (vv * dy + ddy2) + u
                e = jnp.exp(q)
                bb = (r - pr0) * _P + off
                tt = t[pl.ds(bb, 16)]
                w = e * tt
                t[pl.ds(bb, 16)] = tt - w
                plsc.addupdate(acc.at[pl.ds(bb, 16)], cr * w)
                plsc.addupdate(acc.at[pl.ds(bb + 1024, 16)], cg * w)
                plsc.addupdate(acc.at[pl.ds(bb + 2048, 16)], cb * w)
                plsc.addupdate(acc.at[pl.ds(bb + 3072, 16)], w)
                return rc

            return row

        @pl.when(inc0)
        def _():
            lax.fori_loop(ylo, yhi, make_row(x0v, 0), 0)

        @pl.when(inc1)
        def _():
            lax.fori_loop(ylo, yhi, make_row(x1v, 16), 0)

        return c

    lax.fori_loop(0, cnt, hit_body, 0)

    pltpu.sync_copy(acc, out_hbm.at[wid])


@jax.jit
def kernel(batch_ids, means, sigmas, rhos, colors, opacs):
    c = jnp.cos(rhos[:, 0])
    s = jnp.sin(rhos[:, 0])
    s1 = sigmas[:, 0]
    s2 = sigmas[:, 1]
    i1 = 1.0 / s1
    i2 = 1.0 / s2
    a = -0.5 * (c * c * i1 + s * s * i2)
    d = -0.5 * (s * s * i1 + c * c * i2)
    bc = -0.5 * (2.0 * c * s * (i1 - i2))
    sxx = c * c * s1 + s * s * s2
    syy = s * s * s1 + c * c * s2
    rx = jnp.sqrt(_QMAX * sxx)
    ry = jnp.sqrt(_QMAX * syy)
    bidf = batch_ids.astype(jnp.float32)
    lnop = jnp.log(opacs[:, 0])
    z = jnp.zeros_like(a)
    pk = jnp.stack([
        bidf, means[:, 0], means[:, 1],
        a, bc, d, colors[:, 0], colors[:, 1], colors[:, 2],
        lnop, ry, rx, z, z, z, z,
    ], axis=1).reshape(-1)
    cull = jnp.concatenate([bidf, means[:, 1], ry, means[:, 0], rx])
    out = _sc_render(pk, cull)
    out = out.reshape(_B, 4, 4, 4, _P, _P)
    out = out.transpose(0, 3, 1, 4, 2, 5).reshape(_B, 4, _H, _W)
    return out


# speedup vs baseline: 2.9188x; 1.1915x over previous
import functools

import jax
import jax.numpy as jnp
from jax import lax
from jax.experimental import pallas as pl
from jax.experimental.pallas import tpu as pltpu
from jax.experimental.pallas import tpu_sc as plsc

_H = 128
_W = 128
_B = 2
_N = 1024
_P = 32
_QMAX = 30.0

_mesh = plsc.VectorSubcoreMesh(core_axis_name="c", subcore_axis_name="s")


@functools.partial(
    pl.kernel,
    out_type=jax.ShapeDtypeStruct((32, 4 * _P * _P), jnp.float32),
    mesh=_mesh,
    scratch_types=[
        pltpu.VMEM((_N * 16,), jnp.float32),
        pltpu.VMEM((5 * _N,), jnp.float32),
        pltpu.SMEM((_N + 1,), jnp.int32),
        pltpu.VMEM((16,), jnp.int32),
        pltpu.VMEM((_P * _P,), jnp.float32),
        pltpu.VMEM((4 * _P * _P,), jnp.float32),
    ],
)
def _sc_render(pk_hbm, cull_hbm, out_hbm, pk, cull, hits, hvbuf, t, acc):
    wid = lax.axis_index("s") * 2 + lax.axis_index("c")
    b = wid // 16
    pidx = wid % 16
    pr0 = (pidx // 4) * _P
    pc0 = (pidx % 4) * _P
    pr0f = pr0.astype(jnp.float32)
    pc0f = pc0.astype(jnp.float32)
    myb = b.astype(jnp.float32)

    pltpu.sync_copy(pk_hbm, pk)
    pltpu.sync_copy(cull_hbm, cull)

    ones = jnp.full((16,), 1.0, jnp.float32)
    zeros = jnp.zeros((16,), jnp.float32)

    def init_t(j, c):
        t[pl.ds(j * 16, 16)] = ones
        return c

    lax.fori_loop(0, (_P * _P) // 16, init_t, 0)

    def init_acc(j, c):
        acc[pl.ds(j * 16, 16)] = zeros
        return c

    lax.fori_loop(0, (4 * _P * _P) // 16, init_acc, 0)

    lanei = lax.iota(jnp.int32, 16)


    lane = lanei.astype(jnp.float32) + 0.5
    x0v = lane + pc0f
    x1v = x0v + 16.0

    def cull_body(j, cnt):
        j16 = j * 16
        bidv = cull[pl.ds(j16, 16)]
        myv = cull[pl.ds(j16 + _N, 16)]
        ryv = cull[pl.ds(j16 + 2 * _N, 16)]
        mxv = cull[pl.ds(j16 + 3 * _N, 16)]
        rxv = cull[pl.ds(j16 + 4 * _N, 16)]
        hit = bidv == myb
        hit = jnp.logical_and(hit, myv - ryv - 0.5 <= pr0f + (_P - 1.0))
        hit = jnp.logical_and(hit, myv + ryv + 0.5 >= pr0f)
        hit = jnp.logical_and(hit, mxv - rxv - 0.5 < pc0f + float(_P))
        hit = jnp.logical_and(hit, mxv + rxv + 0.5 > pc0f)
        one16 = jnp.full((16,), 1, jnp.int32)
        zero16 = jnp.full((16,), 0, jnp.int32)
        hvbuf[pl.ds(0, 16)] = jnp.where(hit, one16, zero16)
        hb = hvbuf[pl.ds(0, 16)]
        base = j * 16
        for l in range(16):
            hits[cnt] = base + l
            cnt = cnt + hb[l]
        return cnt

    cnt = lax.fori_loop(0, _N // 16, cull_body, 0)

    def hit_body(k, c):
        i = hits[k]
        v = pk[pl.ds(i * 16, 16)]
        mx = v[1]
        my = v[2]
        a = v[3]
        bc = v[4]
        d = v[5]
        cr = v[6]
        cg = v[7]
        cb = v[8]
        lnop = v[9]
        ry = v[10]
        rx = v[11]
        ylo = jnp.maximum(my - ry - 0.5, pr0f).astype(jnp.int32)
        yhi = jnp.minimum(my + ry + 0.5, pr0f + (_P - 1.0)).astype(jnp.int32) + 1
        inc0 = mx - rx - 0.5 < pc0f + 16.0
        inc1 = mx + rx + 0.5 > pc0f + 16.0

        def make_row(xv, off):
            dxh = xv - mx
            u = a * dxh * dxh
            vv = bc * dxh

            def row(r, rc):
                dy = r.astype(jnp.float32) + 0.5 - my
                ddy2 = d * dy * dy + lnop
                q = (vv * dy + ddy2) + u
                e = jnp.exp(q)
                bb = (r - pr0) * _P + off
                tt = t[pl.ds(bb, 16)]
                w = e * tt
                t[pl.ds(bb, 16)] = tt - w
                plsc.addupdate(acc.at[pl.ds(bb, 16)], cr * w)
                plsc.addupdate(acc.at[pl.ds(bb + 1024, 16)], cg * w)
                plsc.addupdate(acc.at[pl.ds(bb + 2048, 16)], cb * w)
                plsc.addupdate(acc.at[pl.ds(bb + 3072, 16)], w)
                return rc

            return row

        @pl.when(inc0)
        def _():
            body0 = make_row(x0v, 0)

            @plsc.parallel_loop(ylo, yhi, unroll=2)
            def _(r):
                body0(r, 0)

        @pl.when(inc1)
        def _():
            body1 = make_row(x1v, 16)

            @plsc.parallel_loop(ylo, yhi, unroll=2)
            def _(r):
                body1(r, 0)

        return c

    lax.fori_loop(0, cnt, hit_body, 0)

    pltpu.sync_copy(acc, out_hbm.at[wid])


@jax.jit
def kernel(batch_ids, means, sigmas, rhos, colors, opacs):
    c = jnp.cos(rhos[:, 0])
    s = jnp.sin(rhos[:, 0])
    s1 = sigmas[:, 0]
    s2 = sigmas[:, 1]
    i1 = 1.0 / s1
    i2 = 1.0 / s2
    a = -0.5 * (c * c * i1 + s * s * i2)
    d = -0.5 * (s * s * i1 + c * c * i2)
    bc = -0.5 * (2.0 * c * s * (i1 - i2))
    sxx = c * c * s1 + s * s * s2
    syy = s * s * s1 + c * c * s2
    rx = jnp.sqrt(_QMAX * sxx)
    ry = jnp.sqrt(_QMAX * syy)
    bidf = batch_ids.astype(jnp.float32)
    lnop = jnp.log(opacs[:, 0])
    z = jnp.zeros_like(a)
    pk = jnp.stack([
        bidf, means[:, 0], means[:, 1],
        a, bc, d, colors[:, 0], colors[:, 1], colors[:, 2],
        lnop, ry, rx, z, z, z, z,
    ], axis=1).reshape(-1)
    cull = jnp.concatenate([bidf, means[:, 1], ry, means[:, 0], rx])
    out = _sc_render(pk, cull)
    out = out.reshape(_B, 4, 4, 4, _P, _P)
    out = out.transpose(0, 3, 1, 4, 2, 5).reshape(_B, 4, _H, _W)
    return out
